# baseline (speedup 1.0000x reference)
import jax
import jax.numpy as jnp
from jax.experimental import pallas as pl
from jax.experimental.pallas import tpu as pltpu

N = 10000
F_IN = 128
F_H = 32
N_GRAPHS = 64
BLK = 1000
GRID = N // BLK
BPAD = ((BLK + 127) // 128) * 128


def _tc_body(x_ref, b_ref, w4_ref, bz_ref, bh_ref, wl_ref, bl_ref,
             out_ref, acc_ref):
    i = pl.program_id(0)

    @pl.when(i == 0)
    def _init():
        acc_ref[...] = jnp.zeros_like(acc_ref)

    xb = x_ref[...]
    w4 = w4_ref[0:F_IN, :]
    wz = w4[:, 0:F_H] + w4[:, F_H:2 * F_H]
    wh = w4[:, 2 * F_H:3 * F_H] + w4[:, 3 * F_H:]
    wcat = jnp.concatenate([wz * 0.5, wh], axis=1)
    bcat = jnp.concatenate([bz_ref[...] * 0.5, bh_ref[...]], axis=1)
    g = jnp.dot(xb, wcat, preferred_element_type=jnp.float32) + bcat
    t = jnp.tanh(g)
    s = 0.5 - 0.5 * t[:, 0:F_H]
    hr = jnp.maximum(s * t[:, F_H:], 0.0)
    h = jnp.dot(hr, wl_ref[...],
                preferred_element_type=jnp.float32) + bl_ref[...]

    h2 = jnp.concatenate([h, jnp.ones_like(h)], axis=1)
    b_row = b_ref[0, :, 0:BLK]
    seg = jax.lax.broadcasted_iota(jnp.int32, (N_GRAPHS, BLK), 0)
    mask = (b_row == seg).astype(jnp.float32)
    acc_ref[...] += jnp.dot(mask, h2, preferred_element_type=jnp.float32)

    @pl.when(i == GRID - 1)
    def _fin():
        st = acc_ref[...].T
        out_ref[...] = st[0:1, :] / jnp.maximum(st[1:2, :], 1.0)


def kernel(x, edge_index, edge_weight, batch, Wz0, Wz1, bz, Wr0, Wr1, br,
           Wh0, Wh1, bh, Wl, bl):
    del edge_index, edge_weight, Wr0, Wr1, br
    bp = jnp.pad(batch.reshape(GRID, BLK), ((0, 0), (0, BPAD - BLK)),
                 constant_values=N_GRAPHS).reshape(GRID, 1, BPAD)
    w4 = jnp.concatenate([Wz0, Wz1, Wh0, Wh1], axis=1)
    full = lambda i: (0, 0)
    out = pl.pallas_call(
        _tc_body,
        grid=(GRID,),
        in_specs=[
            pl.BlockSpec((BLK, F_IN), lambda i: (i, 0)),
            pl.BlockSpec((1, 1, BPAD), lambda i: (i, 0, 0)),
            pl.BlockSpec((F_IN + F_H, 4 * F_H), full),
            pl.BlockSpec((1, F_H), full),
            pl.BlockSpec((1, F_H), full),
            pl.BlockSpec((F_H, 1), full),
            pl.BlockSpec((1, 1), full),
        ],
        out_specs=pl.BlockSpec((1, N_GRAPHS), full),
        out_shape=jax.ShapeDtypeStruct((1, N_GRAPHS), jnp.float32),
        scratch_shapes=[pltpu.VMEM((N_GRAPHS, 2), jnp.float32)],
    )(x, bp, w4, bz.reshape(1, F_H), bh.reshape(1, F_H),
      Wl, bl.reshape(1, 1))
    return out.reshape(N_GRAPHS, 1)


# speedup vs baseline: 1.3428x; 1.3428x over previous
import jax
import jax.numpy as jnp
from jax.experimental import pallas as pl
from jax.experimental.pallas import tpu as pltpu

N = 10000
F_IN = 128
F_H = 32
N_GRAPHS = 64
BLK = 5000
GRID = N // BLK
BPAD = ((BLK + 127) // 128) * 128


def _tc_body(x_ref, b_ref, w4_ref, bz_ref, bh_ref, wl_ref, bl_ref,
             out_ref, acc_ref):
    i = pl.program_id(0)

    @pl.when(i == 0)
    def _init():
        acc_ref[...] = jnp.zeros_like(acc_ref)

    xb = x_ref[...]
    w4 = w4_ref[0:F_IN, :]
    wz = w4[:, 0:F_H] + w4[:, F_H:2 * F_H]
    wh = w4[:, 2 * F_H:3 * F_H] + w4[:, 3 * F_H:]
    wcat = jnp.concatenate([wz * 0.5, wh], axis=1)
    bcat = jnp.concatenate([bz_ref[...] * 0.5, bh_ref[...]], axis=1)
    g = jnp.dot(xb, wcat, preferred_element_type=jnp.float32) + bcat
    t = jnp.tanh(g)
    s = 0.5 - 0.5 * t[:, 0:F_H]
    hr = jnp.maximum(s * t[:, F_H:], 0.0)
    h = jnp.dot(hr, wl_ref[...],
                preferred_element_type=jnp.float32) + bl_ref[...]

    h2 = jnp.concatenate([h, jnp.ones_like(h)], axis=1)
    b_row = b_ref[0, :, 0:BLK]
    seg = jax.lax.broadcasted_iota(jnp.int32, (N_GRAPHS, BLK), 0)
    mask = (b_row == seg).astype(jnp.float32)
    acc_ref[...] += jnp.dot(mask, h2, preferred_element_type=jnp.float32)

    @pl.when(i == GRID - 1)
    def _fin():
        st = acc_ref[...].T
        out_ref[...] = st[0:1, :] / jnp.maximum(st[1:2, :], 1.0)


def kernel(x, edge_index, edge_weight, batch, Wz0, Wz1, bz, Wr0, Wr1, br,
           Wh0, Wh1, bh, Wl, bl):
    del edge_index, edge_weight, Wr0, Wr1, br
    bp = jnp.pad(batch.reshape(GRID, BLK), ((0, 0), (0, BPAD - BLK)),
                 constant_values=N_GRAPHS).reshape(GRID, 1, BPAD)
    w4 = jnp.concatenate([Wz0, Wz1, Wh0, Wh1], axis=1)
    full = lambda i: (0, 0)
    out = pl.pallas_call(
        _tc_body,
        grid=(GRID,),
        in_specs=[
            pl.BlockSpec((BLK, F_IN), lambda i: (i, 0)),
            pl.BlockSpec((1, 1, BPAD), lambda i: (i, 0, 0)),
            pl.BlockSpec((F_IN + F_H, 4 * F_H), full),
            pl.BlockSpec((1, F_H), full),
            pl.BlockSpec((1, F_H), full),
            pl.BlockSpec((F_H, 1), full),
            pl.BlockSpec((1, 1), full),
        ],
        out_specs=pl.BlockSpec((1, N_GRAPHS), full),
        out_shape=jax.ShapeDtypeStruct((1, N_GRAPHS), jnp.float32),
        scratch_shapes=[pltpu.VMEM((N_GRAPHS, 2), jnp.float32)],
    )(x, bp, w4, bz.reshape(1, F_H), bh.reshape(1, F_H),
      Wl, bl.reshape(1, 1))
    return out.reshape(N_GRAPHS, 1)


# speedup vs baseline: 1.4739x; 1.0977x over previous
import jax
import jax.numpy as jnp
from jax.experimental import pallas as pl
from jax.experimental.pallas import tpu as pltpu

N = 10000
F_IN = 128
F_H = 32
N_GRAPHS = 64
BLK = 10000
GRID = N // BLK
BPAD = ((BLK + 127) // 128) * 128


def _tc_body(x_ref, b_ref, w4_ref, bz_ref, bh_ref, wl_ref, bl_ref,
             out_ref, acc_ref):
    i = pl.program_id(0)

    @pl.when(i == 0)
    def _init():
        acc_ref[...] = jnp.zeros_like(acc_ref)

    xb = x_ref[...]
    w4 = w4_ref[0:F_IN, :]
    wz = w4[:, 0:F_H] + w4[:, F_H:2 * F_H]
    wh = w4[:, 2 * F_H:3 * F_H] + w4[:, 3 * F_H:]
    wcat = jnp.concatenate([wz * 0.5, wh], axis=1)
    bcat = jnp.concatenate([bz_ref[...] * 0.5, bh_ref[...]], axis=1)
    g = jnp.dot(xb, wcat, preferred_element_type=jnp.float32) + bcat
    t = jnp.tanh(g)
    s = 0.5 - 0.5 * t[:, 0:F_H]
    hr = jnp.maximum(s * t[:, F_H:], 0.0)
    h = jnp.dot(hr, wl_ref[...],
                preferred_element_type=jnp.float32) + bl_ref[...]

    h2 = jnp.concatenate([h, jnp.ones_like(h)], axis=1)
    b_row = b_ref[0, :, 0:BLK]
    seg = jax.lax.broadcasted_iota(jnp.int32, (N_GRAPHS, BLK), 0)
    mask = (b_row == seg).astype(jnp.float32)
    acc_ref[...] += jnp.dot(mask, h2, preferred_element_type=jnp.float32)

    @pl.when(i == GRID - 1)
    def _fin():
        st = acc_ref[...].T
        out_ref[...] = st[0:1, :] / jnp.maximum(st[1:2, :], 1.0)


def kernel(x, edge_index, edge_weight, batch, Wz0, Wz1, bz, Wr0, Wr1, br,
           Wh0, Wh1, bh, Wl, bl):
    del edge_index, edge_weight, Wr0, Wr1, br
    bp = jnp.pad(batch.reshape(GRID, BLK), ((0, 0), (0, BPAD - BLK)),
                 constant_values=N_GRAPHS).reshape(GRID, 1, BPAD)
    w4 = jnp.concatenate([Wz0, Wz1, Wh0, Wh1], axis=1)
    full = lambda i: (0, 0)
    out = pl.pallas_call(
        _tc_body,
        grid=(GRID,),
        in_specs=[
            pl.BlockSpec((BLK, F_IN), lambda i: (i, 0)),
            pl.BlockSpec((1, 1, BPAD), lambda i: (i, 0, 0)),
            pl.BlockSpec((F_IN + F_H, 4 * F_H), full),
            pl.BlockSpec((1, F_H), full),
            pl.BlockSpec((1, F_H), full),
            pl.BlockSpec((F_H, 1), full),
            pl.BlockSpec((1, 1), full),
        ],
        out_specs=pl.BlockSpec((1, N_GRAPHS), full),
        out_shape=jax.ShapeDtypeStruct((1, N_GRAPHS), jnp.float32),
        scratch_shapes=[pltpu.VMEM((N_GRAPHS, 2), jnp.float32)],
    )(x, bp, w4, bz.reshape(1, F_H), bh.reshape(1, F_H),
      Wl, bl.reshape(1, 1))
    return out.reshape(N_GRAPHS, 1)


# speedup vs baseline: 1.4869x; 1.0088x over previous
import jax
import jax.numpy as jnp
from jax.experimental import pallas as pl

N = 10000
F_IN = 128
F_H = 32
N_GRAPHS = 64


def _tc_body(x_ref, b_ref, w4_ref, bz_ref, bh_ref, wl_ref, bl_ref, out_ref):
    xb = x_ref[...]
    w4 = w4_ref[0:F_IN, :]
    wz = w4[:, 0:F_H] + w4[:, F_H:2 * F_H]
    wh = w4[:, 2 * F_H:3 * F_H] + w4[:, 3 * F_H:]
    wcat = jnp.concatenate([wz * 0.5, wh], axis=1)
    bcat = jnp.concatenate([bz_ref[...] * 0.5, bh_ref[...]], axis=1)
    g = jnp.dot(xb, wcat, preferred_element_type=jnp.float32) + bcat
    t = jnp.tanh(g)
    s = 0.5 - 0.5 * t[:, 0:F_H]
    hr = jnp.maximum(s * t[:, F_H:], 0.0)
    hr2 = jnp.concatenate([hr, jnp.ones((N, 1), jnp.float32)], axis=1)

    b_row = b_ref[0, :, :]
    seg = jax.lax.broadcasted_iota(jnp.int32, (N_GRAPHS, N), 0)
    mask = (b_row == seg).astype(jnp.float32)
    st = jnp.dot(mask, hr2, preferred_element_type=jnp.float32)

    sums32 = st[:, 0:F_H]
    cnt = st[:, F_H:F_H + 1]
    num = jnp.dot(sums32, wl_ref[...],
                  preferred_element_type=jnp.float32) + bl_ref[...] * cnt
    out_ref[...] = (num / jnp.maximum(cnt, 1.0)).T


def kernel(x, edge_index, edge_weight, batch, Wz0, Wz1, bz, Wr0, Wr1, br,
           Wh0, Wh1, bh, Wl, bl):
    del edge_index, edge_weight, Wr0, Wr1, br
    bp = batch.reshape(1, 1, N)
    w4 = jnp.concatenate([Wz0, Wz1, Wh0, Wh1], axis=1)
    out = pl.pallas_call(
        _tc_body,
        in_specs=[
            pl.BlockSpec((N, F_IN), lambda: (0, 0)),
            pl.BlockSpec((1, 1, N), lambda: (0, 0, 0)),
            pl.BlockSpec((F_IN + F_H, 4 * F_H), lambda: (0, 0)),
            pl.BlockSpec((1, F_H), lambda: (0, 0)),
            pl.BlockSpec((1, F_H), lambda: (0, 0)),
            pl.BlockSpec((F_H, 1), lambda: (0, 0)),
            pl.BlockSpec((1, 1), lambda: (0, 0)),
        ],
        out_specs=pl.BlockSpec((1, N_GRAPHS), lambda: (0, 0)),
        out_shape=jax.ShapeDtypeStruct((1, N_GRAPHS), jnp.float32),
    )(x, bp, w4, bz.reshape(1, F_H), bh.reshape(1, F_H),
      Wl, bl.reshape(1, 1))
    return out.reshape(N_GRAPHS, 1)
